# baseline (speedup 1.0000x reference)
import functools

import jax
import jax.numpy as jnp
from jax import lax
from jax.experimental import pallas as pl
from jax.experimental.pallas import tpu as pltpu
from jax.experimental.pallas import tpu_sc as plsc

DIM = 32
BR = 8
LANES = 16


def _make_detile(vocab: int):
    info = plsc.get_sparse_core_info()
    nc, ns = info.num_cores, info.num_subcores
    nw = nc * ns
    n_full = vocab // 128
    tail = vocab - n_full * 128
    per_w = n_full // nw
    n_extra = n_full - per_w * nw
    assert per_w >= 2

    mesh = plsc.VectorSubcoreMesh(core_axis_name="c", subcore_axis_name="s")

    @functools.partial(
        pl.kernel,
        out_type=jax.ShapeDtypeStruct((vocab * DIM,), jnp.float32),
        mesh=mesh,
        scratch_types=[
            pltpu.VMEM((2, DIM, 128), jnp.float32),
            pltpu.VMEM((2, 128 * DIM), jnp.float32),
            pltpu.SemaphoreType.DMA,
            pltpu.SemaphoreType.DMA,
            pltpu.SemaphoreType.DMA,
            pltpu.SemaphoreType.DMA,
        ],
        compiler_params=pltpu.CompilerParams(
            use_tc_tiling_on_sc=True, needs_layout_passes=False),
    )
    def detile_kernel(tt_hbm, tail_hbm, out_hbm, in_v, out_v,
                      si0, si1, so0, so1):
        wid = lax.axis_index("s") * nc + lax.axis_index("c")
        sem_i, sem_o = (si0, si1), (so0, so1)

        def fire_in(blk, b):
            pltpu.async_copy(
                tt_hbm.at[:, pl.ds(blk * 128, 128)], in_v.at[b], sem_i[b])

        def wait_in(b):
            pltpu.make_async_copy(
                tt_hbm.at[:, pl.ds(0, 128)], in_v.at[b], sem_i[b]).wait()

        def fire_out(blk, b):
            pltpu.async_copy(
                out_v.at[b], out_hbm.at[pl.ds(blk * (128 * DIM), 128 * DIM)],
                sem_o[b])

        def wait_out(b):
            pltpu.make_async_copy(
                out_v.at[b], out_hbm.at[pl.ds(0, 128 * DIM)], sem_o[b]).wait()

        def transpose(b):
            c0 = lax.iota(jnp.int32, LANES)
            c1 = c0 + LANES
            for j in range(128):
                jv = jnp.full((LANES,), j, jnp.int32)
                r0 = plsc.load_gather(in_v.at[b], [c0, jv])
                r1 = plsc.load_gather(in_v.at[b], [c1, jv])
                out_v[b, pl.ds(j * DIM, LANES)] = r0
                out_v[b, pl.ds(j * DIM + LANES, LANES)] = r1

        base = wid * per_w

        def step(k, b, first, last):
            nb = 1 - b
            if not last:
                fire_in(base + k + 1, nb)
            wait_in(b)
            if not first:
                wait_out(b)
            transpose(b)
            fire_out(base + k, b)

        fire_in(base, 0)
        step(0, 0, True, False)
        step(1, 1, True, False)

        def body(i, carry):
            step(2 * i, 0, False, False)
            step(2 * i + 1, 1, False, False)
            return carry

        lax.fori_loop(1, per_w // 2 - 1, body, 0)

        step(per_w - 2, 0, False, False)
        step(per_w - 1, 1, False, True)
        wait_out(0)
        wait_out(1)

        eblk = nw * per_w + wid

        @pl.when(wid < n_extra)
        def _():
            pltpu.async_copy(
                tt_hbm.at[:, pl.ds(eblk * 128, 128)], in_v.at[0], sem_i[0])
            pltpu.make_async_copy(
                tt_hbm.at[:, pl.ds(0, 128)], in_v.at[0], sem_i[0]).wait()
            transpose(0)
            pltpu.async_copy(
                out_v.at[0], out_hbm.at[pl.ds(eblk * (128 * DIM), 128 * DIM)],
                sem_o[0])
            wait_out(0)

        if tail:
            @pl.when(wid == n_extra)
            def _():
                dst = out_v.at[0].at[pl.ds(0, tail * DIM)]
                pltpu.async_copy(tail_hbm, dst, sem_i[0])
                pltpu.make_async_copy(tail_hbm, dst, sem_i[0]).wait()
                pltpu.async_copy(
                    dst,
                    out_hbm.at[pl.ds(n_full * 128 * DIM, tail * DIM)],
                    sem_o[0])
                pltpu.make_async_copy(
                    dst, out_hbm.at[pl.ds(0, tail * DIM)], sem_o[0]).wait()

    return detile_kernel


def _make_gather(nb: int, nt: int, vocab: int):
    info = plsc.get_sparse_core_info()
    nc, ns = info.num_cores, info.num_subcores
    nw = nc * ns
    rows_w = nb // nw
    n_chunks = rows_w // BR
    assert nb % nw == 0 and rows_w % BR == 0 and n_chunks % 2 == 0
    assert n_chunks >= 6
    splits = []
    off = 0
    while off < nt:
        w = min(128, nt - off)
        splits.append((off, w))
        off += w
    assert all(o % 8 == 0 for o, _ in splits)

    mesh = plsc.VectorSubcoreMesh(core_axis_name="c", subcore_axis_name="s")

    @functools.partial(
        pl.kernel,
        out_type=jax.ShapeDtypeStruct((nb, nt, DIM), jnp.float32),
        mesh=mesh,
        scratch_types=[
            pltpu.VMEM((2, BR, nt), jnp.int32),
            pltpu.VMEM((2, BR, nt, DIM), jnp.float32),
            pltpu.SemaphoreType.DMA,
            pltpu.SemaphoreType.DMA,
            pltpu.SemaphoreType.DMA,
            pltpu.SemaphoreType.DMA,
            pltpu.SemaphoreType.DMA,
            pltpu.SemaphoreType.DMA,
        ],
        compiler_params=pltpu.CompilerParams(use_tc_tiling_on_sc=False),
    )
    def gather_kernel(table_hbm, idx_hbm, out_hbm, idx_v, rows_v,
                      si0, sg0, sw0, si1, sg1, sw1):
        wid = lax.axis_index("s") * nc + lax.axis_index("c")
        row0 = wid * rows_w
        sem_i, sem_g, sem_w = (si0, si1), (sg0, sg1), (sw0, sw1)

        def fire_idx(c, b):
            pltpu.async_copy(
                idx_hbm.at[pl.ds(row0 + c * BR, BR)], idx_v.at[b], sem_i[b])

        def wait_idx(b):
            pltpu.make_async_copy(
                idx_hbm.at[pl.ds(row0, BR)], idx_v.at[b], sem_i[b]).wait()

        def fire_gathers(c, b):
            for j in range(BR):
                for off, w in splits:
                    pltpu.async_copy(
                        table_hbm.at[idx_v.at[b].at[j].at[pl.ds(off, w)]],
                        rows_v.at[b].at[j].at[pl.ds(off, w)],
                        sem_g[b])

        def wait_gathers(b):
            pltpu.make_async_copy(
                table_hbm.at[pl.ds(0, BR * nt)],
                rows_v.at[b], sem_g[b]).wait()

        def fire_wb(c, b):
            pltpu.async_copy(
                rows_v.at[b], out_hbm.at[pl.ds(row0 + c * BR, BR)], sem_w[b])

        def wait_wb(b):
            pltpu.make_async_copy(
                rows_v.at[b], out_hbm.at[pl.ds(row0, BR)], sem_w[b]).wait()

        def step(g, b, first, last):
            nb_ = 1 - b
            if not last or b == 0:
                wait_idx(nb_)
                if not first or b == 1:
                    wait_wb(nb_)
                fire_gathers(g + 1, nb_)
            wait_gathers(b)
            fire_wb(g, b)
            if not last:
                fire_idx(g + 2, b)

        fire_idx(0, 0)
        fire_idx(1, 1)
        wait_idx(0)
        fire_gathers(0, 0)

        step(0, 0, True, False)
        step(1, 1, True, False)

        def body(i, carry):
            step(2 * i, 0, False, False)
            step(2 * i + 1, 1, False, False)
            return carry

        lax.fori_loop(1, n_chunks // 2 - 1, body, 0)

        g_last = n_chunks - 2
        step(g_last, 0, False, True)
        step(g_last + 1, 1, False, True)

        wait_wb(0)
        wait_wb(1)

    return gather_kernel


def kernel(indices, table):
    nb, nt = indices.shape
    vocab, dim = table.shape
    n_full = vocab // 128
    tail_flat = table[n_full * 128:].reshape(-1)
    flat = _make_detile(vocab)(table.T, tail_flat)
    table_rm = flat.reshape(vocab, dim)
    return _make_gather(nb, nt, vocab)(table_rm, indices.astype(jnp.int32))


# speedup vs baseline: 1.6246x; 1.6246x over previous
import functools

import jax
import jax.numpy as jnp
from jax import lax
from jax.experimental import pallas as pl
from jax.experimental.pallas import tpu as pltpu
from jax.experimental.pallas import tpu_sc as plsc

DIM = 32
BR = 8
LANES = 16


def _make_detile(vocab: int):
    info = plsc.get_sparse_core_info()
    nc, ns = info.num_cores, info.num_subcores
    nw = nc * ns
    n_full = vocab // 128
    tail = vocab - n_full * 128
    per_w = n_full // nw
    n_extra = n_full - per_w * nw
    assert per_w >= 2

    mesh = plsc.VectorSubcoreMesh(core_axis_name="c", subcore_axis_name="s")

    @functools.partial(
        pl.kernel,
        out_type=jax.ShapeDtypeStruct((vocab * DIM,), jnp.float32),
        mesh=mesh,
        scratch_types=[
            pltpu.VMEM((DIM, 128), jnp.float32),
            pltpu.VMEM((DIM, 128), jnp.float32),
            pltpu.VMEM((128 * DIM,), jnp.float32),
            pltpu.VMEM((128 * DIM,), jnp.float32),
            pltpu.SemaphoreType.DMA,
            pltpu.SemaphoreType.DMA,
            pltpu.SemaphoreType.DMA,
            pltpu.SemaphoreType.DMA,
        ],
        compiler_params=pltpu.CompilerParams(
            use_tc_tiling_on_sc=True, needs_layout_passes=False),
    )
    def detile_kernel(tt_hbm, tail_hbm, out_hbm, in_v0, in_v1,
                      out_v0, out_v1, si0, si1, so0, so1):
        wid = lax.axis_index("s") * nc + lax.axis_index("c")
        sem_i, sem_o = (si0, si1), (so0, so1)
        in_v, out_v = (in_v0, in_v1), (out_v0, out_v1)

        def fire_in(blk, b):
            pltpu.async_copy(
                tt_hbm.at[:, pl.ds(blk * 128, 128)], in_v[b], sem_i[b])

        def wait_in(b):
            pltpu.make_async_copy(
                tt_hbm.at[:, pl.ds(0, 128)], in_v[b], sem_i[b]).wait()

        def fire_out(blk, b):
            pltpu.async_copy(
                out_v[b], out_hbm.at[pl.ds(blk * (128 * DIM), 128 * DIM)],
                sem_o[b])

        def wait_out(b):
            pltpu.make_async_copy(
                out_v[b], out_hbm.at[pl.ds(0, 128 * DIM)], sem_o[b]).wait()

        lane = lax.iota(jnp.int32, LANES)
        cidx = (lane, lane + 16)
        rot = [(lane + d) & 15 for d in range(LANES)]
        oid = [r * DIM + lane for r in rot]

        def transpose(b):
            def tbody(i, carry):
                j0 = i * LANES
                for h in range(2):
                    for d in range(LANES):
                        r = plsc.load_gather(
                            in_v[b], [cidx[h], rot[d] + j0])
                        plsc.store_scatter(
                            out_v[b], [oid[d] + (j0 * DIM + 16 * h)], r)
                return carry
            lax.fori_loop(0, 128 // LANES, tbody, 0)

        base = wid * per_w

        def step(k, b, first, last):
            nb = 1 - b
            if not last:
                fire_in(base + k + 1, nb)
            wait_in(b)
            if not first:
                wait_out(b)
            transpose(b)
            fire_out(base + k, b)

        fire_in(base, 0)
        step(0, 0, True, False)
        step(1, 1, True, False)

        def body(i, carry):
            step(2 * i, 0, False, False)
            step(2 * i + 1, 1, False, False)
            return carry

        lax.fori_loop(1, per_w // 2 - 1, body, 0)

        step(per_w - 2, 0, False, False)
        step(per_w - 1, 1, False, True)
        wait_out(0)
        wait_out(1)

        eblk = nw * per_w + wid

        @pl.when(wid < n_extra)
        def _():
            fire_in(eblk, 0)
            wait_in(0)
            transpose(0)
            fire_out(eblk, 0)
            wait_out(0)

        if tail:
            @pl.when(wid == n_extra)
            def _():
                dst = out_v[0].at[pl.ds(0, tail * DIM)]
                pltpu.async_copy(tail_hbm, dst, sem_i[0])
                pltpu.make_async_copy(tail_hbm, dst, sem_i[0]).wait()
                pltpu.async_copy(
                    dst,
                    out_hbm.at[pl.ds(n_full * 128 * DIM, tail * DIM)],
                    sem_o[0])
                pltpu.make_async_copy(
                    dst, out_hbm.at[pl.ds(0, tail * DIM)], sem_o[0]).wait()

    return detile_kernel


def _make_gather(nb: int, nt: int, vocab: int):
    info = plsc.get_sparse_core_info()
    nc, ns = info.num_cores, info.num_subcores
    nw = nc * ns
    rows_w = nb // nw
    n_chunks = rows_w // BR
    assert nb % nw == 0 and rows_w % BR == 0 and n_chunks % 2 == 0
    assert n_chunks >= 6
    splits = []
    off = 0
    while off < nt:
        w = min(128, nt - off)
        splits.append((off, w))
        off += w
    assert all(o % 8 == 0 for o, _ in splits)

    mesh = plsc.VectorSubcoreMesh(core_axis_name="c", subcore_axis_name="s")

    @functools.partial(
        pl.kernel,
        out_type=jax.ShapeDtypeStruct((nb, nt, DIM), jnp.float32),
        mesh=mesh,
        scratch_types=[
            pltpu.VMEM((2, BR, nt), jnp.int32),
            pltpu.VMEM((2, BR, nt, DIM), jnp.float32),
            pltpu.SemaphoreType.DMA,
            pltpu.SemaphoreType.DMA,
            pltpu.SemaphoreType.DMA,
            pltpu.SemaphoreType.DMA,
            pltpu.SemaphoreType.DMA,
            pltpu.SemaphoreType.DMA,
        ],
        compiler_params=pltpu.CompilerParams(use_tc_tiling_on_sc=False),
    )
    def gather_kernel(table_hbm, idx_hbm, out_hbm, idx_v, rows_v,
                      si0, sg0, sw0, si1, sg1, sw1):
        wid = lax.axis_index("s") * nc + lax.axis_index("c")
        row0 = wid * rows_w
        sem_i, sem_g, sem_w = (si0, si1), (sg0, sg1), (sw0, sw1)

        def fire_idx(c, b):
            pltpu.async_copy(
                idx_hbm.at[pl.ds(row0 + c * BR, BR)], idx_v.at[b], sem_i[b])

        def wait_idx(b):
            pltpu.make_async_copy(
                idx_hbm.at[pl.ds(row0, BR)], idx_v.at[b], sem_i[b]).wait()

        def fire_gathers(c, b):
            for j in range(BR):
                for off, w in splits:
                    pltpu.async_copy(
                        table_hbm.at[idx_v.at[b].at[j].at[pl.ds(off, w)]],
                        rows_v.at[b].at[j].at[pl.ds(off, w)],
                        sem_g[b])

        def wait_gathers(b):
            pltpu.make_async_copy(
                table_hbm.at[pl.ds(0, BR * nt)],
                rows_v.at[b], sem_g[b]).wait()

        def fire_wb(c, b):
            pltpu.async_copy(
                rows_v.at[b], out_hbm.at[pl.ds(row0 + c * BR, BR)], sem_w[b])

        def wait_wb(b):
            pltpu.make_async_copy(
                rows_v.at[b], out_hbm.at[pl.ds(row0, BR)], sem_w[b]).wait()

        def step(g, b, first, last):
            nb_ = 1 - b
            if not last or b == 0:
                wait_idx(nb_)
                if not first or b == 1:
                    wait_wb(nb_)
                fire_gathers(g + 1, nb_)
            wait_gathers(b)
            fire_wb(g, b)
            if not last:
                fire_idx(g + 2, b)

        fire_idx(0, 0)
        fire_idx(1, 1)
        wait_idx(0)
        fire_gathers(0, 0)

        step(0, 0, True, False)
        step(1, 1, True, False)

        def body(i, carry):
            step(2 * i, 0, False, False)
            step(2 * i + 1, 1, False, False)
            return carry

        lax.fori_loop(1, n_chunks // 2 - 1, body, 0)

        g_last = n_chunks - 2
        step(g_last, 0, False, True)
        step(g_last + 1, 1, False, True)

        wait_wb(0)
        wait_wb(1)

    return gather_kernel


def kernel(indices, table):
    nb, nt = indices.shape
    vocab, dim = table.shape
    n_full = vocab // 128
    tail_flat = table[n_full * 128:].reshape(-1)
    flat = _make_detile(vocab)(table.T, tail_flat)
    table_rm = flat.reshape(vocab, dim)
    return _make_gather(nb, nt, vocab)(table_rm, indices.astype(jnp.int32))


# speedup vs baseline: 1.8699x; 1.1510x over previous
import functools

import jax
import jax.numpy as jnp
from jax import lax
from jax.experimental import pallas as pl
from jax.experimental.pallas import tpu as pltpu
from jax.experimental.pallas import tpu_sc as plsc

DIM = 32
BR = 8
LANES = 16


def _make_detile(vocab: int):
    info = plsc.get_sparse_core_info()
    nc, ns = info.num_cores, info.num_subcores
    nw = nc * ns
    n_full = vocab // 128
    tail = vocab - n_full * 128
    per_w = n_full // nw
    n_extra = n_full - per_w * nw
    assert per_w >= 2

    mesh = plsc.VectorSubcoreMesh(core_axis_name="c", subcore_axis_name="s")

    @functools.partial(
        pl.kernel,
        out_type=jax.ShapeDtypeStruct((vocab * DIM,), jnp.float32),
        mesh=mesh,
        scratch_types=[
            pltpu.VMEM((DIM, 128), jnp.float32),
            pltpu.VMEM((DIM, 128), jnp.float32),
            pltpu.VMEM((128 * DIM,), jnp.float32),
            pltpu.VMEM((128 * DIM,), jnp.float32),
            pltpu.SemaphoreType.DMA,
            pltpu.SemaphoreType.DMA,
            pltpu.SemaphoreType.DMA,
            pltpu.SemaphoreType.DMA,
        ],
        compiler_params=pltpu.CompilerParams(
            use_tc_tiling_on_sc=True, needs_layout_passes=False),
    )
    def detile_kernel(tt_hbm, tail_hbm, out_hbm, in_v0, in_v1,
                      out_v0, out_v1, si0, si1, so0, so1):
        wid = lax.axis_index("s") * nc + lax.axis_index("c")
        sem_i, sem_o = (si0, si1), (so0, so1)
        in_v, out_v = (in_v0, in_v1), (out_v0, out_v1)

        def fire_in(blk, b):
            pltpu.async_copy(
                tt_hbm.at[:, pl.ds(blk * 128, 128)], in_v[b], sem_i[b])

        def wait_in(b):
            pltpu.make_async_copy(
                tt_hbm.at[:, pl.ds(0, 128)], in_v[b], sem_i[b]).wait()

        def fire_out(blk, b):
            pltpu.async_copy(
                out_v[b], out_hbm.at[pl.ds(blk * (128 * DIM), 128 * DIM)],
                sem_o[b])

        def wait_out(b):
            pltpu.make_async_copy(
                out_v[b], out_hbm.at[pl.ds(0, 128 * DIM)], sem_o[b]).wait()

        lane = lax.iota(jnp.int32, LANES)
        cidx = (lane, lane + 16)
        rot = [(lane + d) & 15 for d in range(LANES)]
        oid = [r * DIM + lane for r in rot]

        def transpose(b):
            def tbody(i, carry):
                j0 = i * LANES
                for h in range(2):
                    for d in range(LANES):
                        r = plsc.load_gather(
                            in_v[b], [cidx[h], rot[d] + j0])
                        plsc.store_scatter(
                            out_v[b], [oid[d] + (j0 * DIM + 16 * h)], r)
                return carry
            lax.fori_loop(0, 128 // LANES, tbody, 0)

        base = wid * per_w

        def step(k, b, first, last):
            nb = 1 - b
            if not last:
                fire_in(base + k + 1, nb)
            wait_in(b)
            if not first:
                wait_out(b)
            transpose(b)
            fire_out(base + k, b)

        fire_in(base, 0)
        step(0, 0, True, False)
        step(1, 1, True, False)

        def body(i, carry):
            step(2 * i, 0, False, False)
            step(2 * i + 1, 1, False, False)
            return carry

        lax.fori_loop(1, per_w // 2 - 1, body, 0)

        step(per_w - 2, 0, False, False)
        step(per_w - 1, 1, False, True)
        wait_out(0)
        wait_out(1)

        eblk = nw * per_w + wid

        @pl.when(wid < n_extra)
        def _():
            fire_in(eblk, 0)
            wait_in(0)
            transpose(0)
            fire_out(eblk, 0)
            wait_out(0)

        if tail:
            @pl.when(wid == n_extra)
            def _():
                dst = out_v[0].at[pl.ds(0, tail * DIM)]
                pltpu.async_copy(tail_hbm, dst, sem_i[0])
                pltpu.make_async_copy(tail_hbm, dst, sem_i[0]).wait()
                pltpu.async_copy(
                    dst,
                    out_hbm.at[pl.ds(n_full * 128 * DIM, tail * DIM)],
                    sem_o[0])
                pltpu.make_async_copy(
                    dst, out_hbm.at[pl.ds(0, tail * DIM)], sem_o[0]).wait()

    return detile_kernel


def _make_gather(nb: int, nt: int, vocab: int):
    info = plsc.get_sparse_core_info()
    nc, ns = info.num_cores, info.num_subcores
    nw = nc * ns
    bw = nb // nw
    assert nb % nw == 0 and bw == 128
    TT = 4
    n_chunks = nt // TT
    assert nt % TT == 0 and n_chunks % 2 == 0 and n_chunks >= 6

    mesh = plsc.VectorSubcoreMesh(core_axis_name="c", subcore_axis_name="s")

    @functools.partial(
        pl.kernel,
        out_type=jax.ShapeDtypeStruct((nt, DIM, nb), jnp.float32),
        mesh=mesh,
        scratch_types=[
            pltpu.VMEM((TT, 128), jnp.int32),
            pltpu.VMEM((TT, 128), jnp.int32),
            pltpu.VMEM((TT * 128, DIM), jnp.float32),
            pltpu.VMEM((TT * 128, DIM), jnp.float32),
            pltpu.VMEM((TT * DIM, 128), jnp.float32),
            pltpu.VMEM((TT * DIM, 128), jnp.float32),
            pltpu.SemaphoreType.DMA,
            pltpu.SemaphoreType.DMA,
            pltpu.SemaphoreType.DMA,
            pltpu.SemaphoreType.DMA,
            pltpu.SemaphoreType.DMA,
            pltpu.SemaphoreType.DMA,
        ],
        compiler_params=pltpu.CompilerParams(
            use_tc_tiling_on_sc=False, needs_layout_passes=False),
    )
    def gather_kernel(table_hbm, idx_hbm, out_hbm,
                      ix0, ix1, rv0, rv1, tv0, tv1,
                      si0, si1, sg0, sg1, sw0, sw1):
        wid = lax.axis_index("s") * nc + lax.axis_index("c")
        b0 = wid * bw
        idx_v, rows_v, tr_v = (ix0, ix1), (rv0, rv1), (tv0, tv1)
        sem_i, sem_g, sem_w = (si0, si1), (sg0, sg1), (sw0, sw1)

        lane = lax.iota(jnp.int32, LANES)
        rot = [(lane + d) & 15 for d in range(LANES)]
        gcol = [[rot[d] + 16 * h for d in range(LANES)] for h in range(2)]
        gcs = [lane + 16 * g0 for g0 in range(8)]

        def fire_idx(g, b):
            pltpu.async_copy(
                idx_hbm.at[pl.ds(g * TT, TT), pl.ds(b0, 128)],
                idx_v[b], sem_i[b])

        def wait_idx(b):
            pltpu.make_async_copy(
                idx_hbm.at[pl.ds(0, TT), pl.ds(b0, 128)],
                idx_v[b], sem_i[b]).wait()

        def fire_gathers(g, b):
            for t in range(TT):
                pltpu.async_copy(
                    table_hbm.at[idx_v[b].at[t]],
                    rows_v[b].at[pl.ds(t * 128, 128)],
                    sem_g[b])

        def wait_gathers(b):
            pltpu.make_async_copy(
                table_hbm.at[pl.ds(0, TT * 128)],
                rows_v[b], sem_g[b]).wait()

        def transpose(b):
            def tbody(t, carry):
                r0 = t * 128
                s0 = t * DIM
                for g0 in range(8):
                    grow = lane + (r0 + 16 * g0)
                    for h in range(2):
                        for d in range(LANES):
                            r = plsc.load_gather(
                                rows_v[b], [grow, gcol[h][d]])
                            plsc.store_scatter(
                                tr_v[b], [rot[d] + (s0 + 16 * h), gcs[g0]],
                                r)
                return carry
            lax.fori_loop(0, TT, tbody, 0)

        def fire_wb(g, b):
            for t in range(TT):
                pltpu.async_copy(
                    tr_v[b].at[pl.ds(t * DIM, DIM)],
                    out_hbm.at[g * TT + t].at[:, pl.ds(b0, 128)],
                    sem_w[b])

        def wait_wb(b):
            for t in range(TT):
                pltpu.make_async_copy(
                    tr_v[b].at[pl.ds(t * DIM, DIM)],
                    out_hbm.at[0].at[:, pl.ds(b0, 128)], sem_w[b]).wait()

        def step(g, b, first, last):
            nb_ = 1 - b
            if not last or b == 0:
                wait_idx(nb_)
                fire_gathers(g + 1, nb_)
            wait_gathers(b)
            if not first:
                wait_wb(b)
            transpose(b)
            fire_wb(g, b)
            if not last:
                fire_idx(g + 2, b)

        fire_idx(0, 0)
        fire_idx(1, 1)
        wait_idx(0)
        fire_gathers(0, 0)

        step(0, 0, True, False)
        step(1, 1, True, False)

        def body(i, carry):
            step(2 * i, 0, False, False)
            step(2 * i + 1, 1, False, False)
            return carry

        lax.fori_loop(1, n_chunks // 2 - 1, body, 0)

        g_last = n_chunks - 2
        step(g_last, 0, False, True)
        step(g_last + 1, 1, False, True)

        wait_wb(0)
        wait_wb(1)

    return gather_kernel


def kernel(indices, table):
    nb, nt = indices.shape
    vocab, dim = table.shape
    n_full = vocab // 128
    tail_flat = table[n_full * 128:].reshape(-1)
    flat = _make_detile(vocab)(table.T, tail_flat)
    table_rm = flat.reshape(vocab, dim)
    idx_t = indices.T.astype(jnp.int32)
    out_t = _make_gather(nb, nt, vocab)(table_rm, idx_t)
    return jnp.transpose(out_t, (2, 0, 1))


# speedup vs baseline: 2.1992x; 1.1761x over previous
import functools

import jax
import jax.numpy as jnp
from jax import lax
from jax.experimental import pallas as pl
from jax.experimental.pallas import tpu as pltpu
from jax.experimental.pallas import tpu_sc as plsc

DIM = 32
BR = 8
LANES = 16


def _make_detile(vocab: int):
    info = plsc.get_sparse_core_info()
    nc, ns = info.num_cores, info.num_subcores
    nw = nc * ns
    n_full = vocab // 128
    tail = vocab - n_full * 128
    per_w = n_full // nw
    n_extra = n_full - per_w * nw
    assert per_w >= 2

    mesh = plsc.VectorSubcoreMesh(core_axis_name="c", subcore_axis_name="s")

    @functools.partial(
        pl.kernel,
        out_type=jax.ShapeDtypeStruct((vocab * DIM,), jnp.float32),
        mesh=mesh,
        scratch_types=[
            pltpu.VMEM((DIM, 128), jnp.float32),
            pltpu.VMEM((DIM, 128), jnp.float32),
            pltpu.VMEM((128 * DIM,), jnp.float32),
            pltpu.VMEM((128 * DIM,), jnp.float32),
            pltpu.SemaphoreType.DMA,
            pltpu.SemaphoreType.DMA,
            pltpu.SemaphoreType.DMA,
            pltpu.SemaphoreType.DMA,
        ],
        compiler_params=pltpu.CompilerParams(
            use_tc_tiling_on_sc=True, needs_layout_passes=False),
    )
    def detile_kernel(tt_hbm, tail_hbm, out_hbm, in_v0, in_v1,
                      out_v0, out_v1, si0, si1, so0, so1):
        wid = lax.axis_index("s") * nc + lax.axis_index("c")
        sem_i, sem_o = (si0, si1), (so0, so1)
        in_v, out_v = (in_v0, in_v1), (out_v0, out_v1)

        def fire_in(blk, b):
            pltpu.async_copy(
                tt_hbm.at[:, pl.ds(blk * 128, 128)], in_v[b], sem_i[b])

        def wait_in(b):
            pltpu.make_async_copy(
                tt_hbm.at[:, pl.ds(0, 128)], in_v[b], sem_i[b]).wait()

        def fire_out(blk, b):
            pltpu.async_copy(
                out_v[b], out_hbm.at[pl.ds(blk * (128 * DIM), 128 * DIM)],
                sem_o[b])

        def wait_out(b):
            pltpu.make_async_copy(
                out_v[b], out_hbm.at[pl.ds(0, 128 * DIM)], sem_o[b]).wait()

        lane = lax.iota(jnp.int32, LANES)
        cidx = (lane, lane + 16)
        rot = [(lane + d) & 15 for d in range(LANES)]
        oid = [r * DIM + lane for r in rot]

        def transpose(b):
            def tbody(i, carry):
                j0 = i * LANES
                for h in range(2):
                    for d in range(LANES):
                        r = plsc.load_gather(
                            in_v[b], [cidx[h], rot[d] + j0])
                        plsc.store_scatter(
                            out_v[b], [oid[d] + (j0 * DIM + 16 * h)], r)
                return carry
            lax.fori_loop(0, 128 // LANES, tbody, 0)

        base = wid * per_w

        def step(k, b, first, last):
            nb = 1 - b
            if not last:
                fire_in(base + k + 1, nb)
            wait_in(b)
            if not first:
                wait_out(b)
            transpose(b)
            fire_out(base + k, b)

        fire_in(base, 0)
        step(0, 0, True, False)
        step(1, 1, True, False)

        def body(i, carry):
            step(2 * i, 0, False, False)
            step(2 * i + 1, 1, False, False)
            return carry

        lax.fori_loop(1, per_w // 2 - 1, body, 0)

        step(per_w - 2, 0, False, False)
        step(per_w - 1, 1, False, True)
        wait_out(0)
        wait_out(1)

        eblk = nw * per_w + wid

        @pl.when(wid < n_extra)
        def _():
            fire_in(eblk, 0)
            wait_in(0)
            transpose(0)
            fire_out(eblk, 0)
            wait_out(0)

        if tail:
            @pl.when(wid == n_extra)
            def _():
                dst = out_v[0].at[pl.ds(0, tail * DIM)]
                pltpu.async_copy(tail_hbm, dst, sem_i[0])
                pltpu.make_async_copy(tail_hbm, dst, sem_i[0]).wait()
                pltpu.async_copy(
                    dst,
                    out_hbm.at[pl.ds(n_full * 128 * DIM, tail * DIM)],
                    sem_o[0])
                pltpu.make_async_copy(
                    dst, out_hbm.at[pl.ds(0, tail * DIM)], sem_o[0]).wait()

    return detile_kernel


def _make_gather(nb: int, nt: int, vocab: int):
    info = plsc.get_sparse_core_info()
    nc, ns = info.num_cores, info.num_subcores
    nw = nc * ns
    bw = nb // nw
    assert nb % nw == 0 and bw == 128
    TT = 4
    n_chunks = nt // TT
    assert nt % TT == 0 and n_chunks % 2 == 0 and n_chunks >= 6

    mesh = plsc.VectorSubcoreMesh(core_axis_name="c", subcore_axis_name="s")

    @functools.partial(
        pl.kernel,
        out_type=jax.ShapeDtypeStruct((nt, DIM // 8, nb // 128, 8, 128),
                                      jnp.float32),
        mesh=mesh,
        scratch_types=[
            pltpu.VMEM((TT, 128), jnp.int32),
            pltpu.VMEM((TT, 128), jnp.int32),
            pltpu.VMEM((TT * 128, DIM), jnp.float32),
            pltpu.VMEM((TT * 128, DIM), jnp.float32),
            pltpu.VMEM((TT * DIM, 128), jnp.float32),
            pltpu.VMEM((TT * DIM, 128), jnp.float32),
            pltpu.SemaphoreType.DMA,
            pltpu.SemaphoreType.DMA,
            pltpu.SemaphoreType.DMA,
            pltpu.SemaphoreType.DMA,
            pltpu.SemaphoreType.DMA,
            pltpu.SemaphoreType.DMA,
        ],
        compiler_params=pltpu.CompilerParams(
            use_tc_tiling_on_sc=False, needs_layout_passes=False),
    )
    def gather_kernel(table_hbm, idx_hbm, out_hbm,
                      ix0, ix1, rv0, rv1, tv0, tv1,
                      si0, si1, sg0, sg1, sw0, sw1):
        wid = lax.axis_index("s") * nc + lax.axis_index("c")
        b0 = wid * bw
        idx_v, rows_v, tr_v = (ix0, ix1), (rv0, rv1), (tv0, tv1)
        sem_i, sem_g, sem_w = (si0, si1), (sg0, sg1), (sw0, sw1)

        lane = lax.iota(jnp.int32, LANES)
        rot = [(lane + d) & 15 for d in range(LANES)]
        gcol = [[rot[d] + 16 * h for d in range(LANES)] for h in range(2)]
        gcs = [lane + 16 * g0 for g0 in range(8)]

        def fire_idx(g, b):
            pltpu.async_copy(
                idx_hbm.at[pl.ds(g * TT, TT), pl.ds(b0, 128)],
                idx_v[b], sem_i[b])

        def wait_idx(b):
            pltpu.make_async_copy(
                idx_hbm.at[pl.ds(0, TT), pl.ds(b0, 128)],
                idx_v[b], sem_i[b]).wait()

        def fire_gathers(g, b):
            for t in range(TT):
                pltpu.async_copy(
                    table_hbm.at[idx_v[b].at[t]],
                    rows_v[b].at[pl.ds(t * 128, 128)],
                    sem_g[b])

        def wait_gathers(b):
            pltpu.make_async_copy(
                table_hbm.at[pl.ds(0, TT * 128)],
                rows_v[b], sem_g[b]).wait()

        def transpose(b):
            def tbody(t, carry):
                r0 = t * 128
                s0 = t * DIM
                for g0 in range(8):
                    grow = lane + (r0 + 16 * g0)
                    for h in range(2):
                        for d in range(LANES):
                            r = plsc.load_gather(
                                rows_v[b], [grow, gcol[h][d]])
                            plsc.store_scatter(
                                tr_v[b], [rot[d] + (s0 + 16 * h), gcs[g0]],
                                r)
                return carry
            lax.fori_loop(0, TT, tbody, 0)

        def fire_wb(g, b):
            for t in range(TT):
                for cb in range(DIM // 8):
                    pltpu.async_copy(
                        tr_v[b].at[pl.ds(t * DIM + cb * 8, 8)],
                        out_hbm.at[g * TT + t].at[cb].at[wid],
                        sem_w[b])

        def wait_wb(b):
            for t in range(TT):
                for cb in range(DIM // 8):
                    pltpu.make_async_copy(
                        tr_v[b].at[pl.ds(t * DIM + cb * 8, 8)],
                        out_hbm.at[0].at[cb].at[wid], sem_w[b]).wait()

        def step(g, b, first, last):
            nb_ = 1 - b
            if not last or b == 0:
                wait_idx(nb_)
                fire_gathers(g + 1, nb_)
            wait_gathers(b)
            if not first:
                wait_wb(b)
            transpose(b)
            fire_wb(g, b)
            if not last:
                fire_idx(g + 2, b)

        fire_idx(0, 0)
        fire_idx(1, 1)
        wait_idx(0)
        fire_gathers(0, 0)

        step(0, 0, True, False)
        step(1, 1, True, False)

        def body(i, carry):
            step(2 * i, 0, False, False)
            step(2 * i + 1, 1, False, False)
            return carry

        lax.fori_loop(1, n_chunks // 2 - 1, body, 0)

        g_last = n_chunks - 2
        step(g_last, 0, False, True)
        step(g_last + 1, 1, False, True)

        wait_wb(0)
        wait_wb(1)

    return gather_kernel


def kernel(indices, table):
    nb, nt = indices.shape
    vocab, dim = table.shape
    n_full = vocab // 128
    tail_flat = table[n_full * 128:].reshape(-1)
    flat = _make_detile(vocab)(table.T, tail_flat)
    table_rm = flat.reshape(vocab, dim)
    idx_t = indices.T.astype(jnp.int32)
    out5 = _make_gather(nb, nt, vocab)(table_rm, idx_t)
    return jnp.transpose(out5, (2, 4, 0, 1, 3)).reshape(nb, nt, dim)
